# baseline (speedup 1.0000x reference)
import functools

import jax
import jax.numpy as jnp
from jax import lax
from jax.experimental import pallas as pl
from jax.experimental.pallas import tpu as pltpu
from jax.experimental.pallas import tpu_sc as plsc

NU = 50000
NI = 50000
D = 64
H = 32
NNZ = 800000
B = 4096
NSC = 2
NTILE = 16
CHUNK = 128
BCH = 6
NBLKE = 67
NCHUNK = BCH * NBLKE
EPT = CHUNK * NCHUNK
EPAD = EPT * NTILE
EROWS = EPAD // CHUNK
RCH = 128
NRCH = 391
LASTN = NU - (NRCH - 1) * RCH
LASTK = NRCH - 1
NROUND = 24

MBLK = 1000
NBLK = NI // MBLK


def _mesh():
    return plsc.VectorSubcoreMesh(core_axis_name="c", subcore_axis_name="s",
                                  num_cores=NSC, num_subcores=NTILE)


def _prep_body(vem, W, b, w1h, w1l, o_itf, o_h1h, o_h1l):
    x = vem[...]
    itf = jnp.dot(x, W[...], preferred_element_type=jnp.float32) + b[...]
    o_itf[0] = itf[:, :H]
    o_itf[1] = itf[:, H:]
    h1h = jnp.dot(itf, w1h[...], preferred_element_type=jnp.float32)
    o_h1h[0] = h1h[:, :H]
    o_h1h[1] = h1h[:, H:]
    h1l = jnp.dot(itf, w1l[...], preferred_element_type=jnp.float32)
    o_h1l[0] = h1l[:, :H]
    o_h1l[1] = h1l[:, H:]


def _prep(video_emb, W_proj, b_proj, w1h, w1l):
    out = jax.ShapeDtypeStruct((2, NI, H), jnp.float32)
    return pl.pallas_call(
        _prep_body,
        grid=(NBLK,),
        in_specs=[
            pl.BlockSpec((MBLK, D), lambda i: (i, 0)),
            pl.BlockSpec((D, D), lambda i: (0, 0)),
            pl.BlockSpec((1, D), lambda i: (0, 0)),
            pl.BlockSpec((D, D), lambda i: (0, 0)),
            pl.BlockSpec((D, D), lambda i: (0, 0)),
        ],
        out_specs=[pl.BlockSpec((2, MBLK, H), lambda i: (0, i, 0))] * 3,
        out_shape=[out, out, out],
    )(video_emb, W_proj, b_proj, w1h, w1l)


def _h2_body(ih, il, w2h, w2l, oh, ol):
    xh = jnp.concatenate([ih[0], ih[1]], axis=1)
    h2h = jnp.dot(xh, w2h[...], preferred_element_type=jnp.float32)
    oh[0] = h2h[:, :H]
    oh[1] = h2h[:, H:]
    xl = jnp.concatenate([il[0], il[1]], axis=1)
    h2l = jnp.dot(xl, w2l[...], preferred_element_type=jnp.float32)
    ol[0] = h2l[:, :H]
    ol[1] = h2l[:, H:]


def _h2(i1h, i1l, w2h, w2l):
    out = jax.ShapeDtypeStruct((2, NI, H), jnp.float32)
    return pl.pallas_call(
        _h2_body,
        grid=(NBLK,),
        in_specs=[
            pl.BlockSpec((2, MBLK, H), lambda i: (0, i, 0)),
            pl.BlockSpec((2, MBLK, H), lambda i: (0, i, 0)),
            pl.BlockSpec((D, D), lambda i: (0, 0)),
            pl.BlockSpec((D, D), lambda i: (0, 0)),
        ],
        out_specs=[pl.BlockSpec((2, MBLK, H), lambda i: (0, i, 0))] * 2,
        out_shape=[out, out],
    )(i1h, i1l, w2h, w2l)


def _for_tile_rows(s, do):
    for j in range(NROUND):
        do((s + NTILE * j) * RCH, RCH)
    rem = LASTK - NTILE * NROUND

    @pl.when(s < rem)
    def _():
        do((s + NTILE * NROUND) * RCH, RCH, tail=True)

    @pl.when(s == rem)
    def _():
        do(LASTK * RCH, LASTN, tail=True)


def _zero_acc(gbuf, acc, s, sem):
    z = jnp.zeros((16,), jnp.float32)
    for r in range(RCH):
        gbuf[r, pl.ds(0, 16)] = z
        gbuf[r, pl.ds(16, 16)] = z
    ds = []

    def do(r0, n, tail=False):
        if tail:
            pltpu.sync_copy(gbuf.at[pl.ds(0, n)], acc.at[pl.ds(r0, n)])
        else:
            ds.append(pltpu.async_copy(gbuf.at[pl.ds(0, n)],
                                       acc.at[pl.ds(r0, n)], sem))

    _for_tile_rows(s, do)
    for d in ds:
        d.wait()


def _edge_pass(tab_ref, gidx_ref, sidx_ref, vals_ref, bufs, acc, c, s, n_tab):
    idxb, sidb, valb, adjb = bufs[0:4]
    gbs = bufs[4:4 + BCH]
    semg = bufs[4 + BCH:4 + 2 * BCH]
    sems = bufs[4 + 2 * BCH:6 + 2 * BCH]
    semb = bufs[6 + 2 * BCH]
    base = c * n_tab
    row0 = s * NCHUNK

    def block_body(bi, carry):
        r = row0 + bi * BCH
        da = pltpu.async_copy(gidx_ref.at[pl.ds(r, BCH)], idxb, semb)
        db = pltpu.async_copy(sidx_ref.at[pl.ds(r, BCH)], sidb, semb)
        dc = pltpu.async_copy(vals_ref.at[pl.ds(r, BCH)], valb, semb)
        da.wait()
        db.wait()
        dc.wait()
        for j in range(BCH):
            for g in range(CHUNK // 16):
                adjb[j, pl.ds(g * 16, 16)] = (idxb[j, pl.ds(g * 16, 16)]
                                              + base)
        d_g = [pltpu.async_copy(tab_ref.at[adjb.at[j]], gbs[j], semg[j])
               for j in range(BCH)]
        d_s = [None] * BCH
        for j in range(BCH):
            d_g[j].wait()
            gb = gbs[j]
            for g in range(CHUNK // 16):
                vv = valb[j, pl.ds(g * 16, 16)]
                for e in range(16):
                    rr = g * 16 + e
                    sp = vv.at[jnp.full((16,), e, jnp.int32)].get(
                        mode="promise_in_bounds")
                    gb[rr, pl.ds(0, 16)] = gb[rr, pl.ds(0, 16)] * sp
                    gb[rr, pl.ds(16, 16)] = gb[rr, pl.ds(16, 16)] * sp
            d_s[j] = pltpu.async_copy(gb, acc.at[sidb.at[j]], sems[j % 2],
                                      add=True)
        for j in range(BCH):
            d_s[j].wait()
        return carry

    lax.fori_loop(0, NBLKE, block_body, 0)


def _pass_scratch(n_out):
    return (
        [pltpu.VMEM((BCH, CHUNK), jnp.int32),
         pltpu.VMEM((BCH, CHUNK), jnp.int32),
         pltpu.VMEM((BCH, CHUNK), jnp.float32),
         pltpu.VMEM((BCH, CHUNK), jnp.int32)]
        + [pltpu.VMEM((CHUNK, H), jnp.float32)] * BCH
        + [pltpu.SemaphoreType.DMA] * BCH
        + [pltpu.SemaphoreType.DMA] * 2
        + [pltpu.SemaphoreType.DMA]
        + [pltpu.VMEM_SHARED((n_out, H), jnp.float32)]
    )


def _spmm(table, gidx, sidx, vals, n_out):
    n_tab = table.shape[0] // 2

    def body(tab_ref, gidx_ref, sidx_ref, vals_ref, out_ref, *bufs):
        acc = bufs[-1]
        semb = bufs[-2]
        c = lax.axis_index("c")
        s = lax.axis_index("s")
        _zero_acc(bufs[4], acc, s, semb)
        plsc.subcore_barrier()
        _edge_pass(tab_ref, gidx_ref, sidx_ref, vals_ref, bufs[:-1], acc,
                   c, s, n_tab)
        plsc.subcore_barrier()
        ds = []

        def do(r0, n, tail=False):
            if tail:
                pltpu.sync_copy(acc.at[pl.ds(r0, n)],
                                out_ref.at[pl.ds(c * n_out + r0, n)])
            else:
                ds.append(pltpu.async_copy(
                    acc.at[pl.ds(r0, n)],
                    out_ref.at[pl.ds(c * n_out + r0, n)], semb))

        _for_tile_rows(s, do)
        for d in ds:
            d.wait()

    f = pl.kernel(
        body,
        out_type=jax.ShapeDtypeStruct((2 * n_out, H), jnp.float32),
        mesh=_mesh(),
        scratch_types=_pass_scratch(n_out),
        compiler_params=pltpu.CompilerParams(use_tc_tiling_on_sc=False),
    )
    return f(table, gidx, sidx, vals)


def _user_final(h2h, h2l, gxh, sxh, vlh, gxl, sxl, vll, u1h, u1l):
    def body(h2h_ref, h2l_ref, gxh_ref, sxh_ref, vlh_ref, gxl_ref, sxl_ref,
             vll_ref, u1h_ref, u1l_ref, out_ref, *bufs):
        acc = bufs[-1]
        semb = bufs[-2]
        gbuf, bufB, bufC = bufs[4], bufs[5], bufs[6]
        c = lax.axis_index("c")
        s = lax.axis_index("s")
        _zero_acc(gbuf, acc, s, semb)
        plsc.subcore_barrier()
        _edge_pass(h2h_ref, gxh_ref, sxh_ref, vlh_ref, bufs[:-1], acc,
                   c, s, NI)
        _edge_pass(h2l_ref, gxl_ref, sxl_ref, vll_ref, bufs[:-1], acc,
                   c, s, NI)
        plsc.subcore_barrier()

        def do(r0, n, tail=False):
            pltpu.sync_copy(acc.at[pl.ds(r0, n)], gbuf.at[pl.ds(0, n)])
            pltpu.sync_copy(u1h_ref.at[pl.ds(c * NU + r0, n)],
                            bufB.at[pl.ds(0, n)])
            pltpu.sync_copy(u1l_ref.at[pl.ds(c * NU + r0, n)],
                            bufC.at[pl.ds(0, n)])

            def addrow(i, carry):
                for j in (0, 16):
                    gbuf[i, pl.ds(j, 16)] = (gbuf[i, pl.ds(j, 16)]
                                             + bufB[i, pl.ds(j, 16)]
                                             + bufC[i, pl.ds(j, 16)])
                return carry

            lax.fori_loop(0, n, addrow, 0)
            pltpu.sync_copy(gbuf.at[pl.ds(0, n)],
                            out_ref.at[pl.ds(c * NU + r0, n)])

        _for_tile_rows(s, do)

    f = pl.kernel(
        body,
        out_type=jax.ShapeDtypeStruct((2 * NU, H), jnp.float32),
        mesh=_mesh(),
        scratch_types=_pass_scratch(NU),
        compiler_params=pltpu.CompilerParams(use_tc_tiling_on_sc=False),
    )
    return f(h2h, h2l, gxh, sxh, vlh, gxl, sxl, vll, u1h, u1l)


SPT = B // (NSC * NTILE)


def _gather_rows(uf, itf, i1h, i1l, uid, iid):
    def body(uf_ref, itf_ref, i1h_ref, i1l_ref, uid_ref, iid_ref,
             oU0, oU1, oA0, oA1, oB0, oB1, oC0, oC1,
             uidx, tidx, gb0, gb1, sem):
        c = lax.axis_index("c")
        s = lax.axis_index("s")
        wid = s * NSC + c
        base = wid * SPT

        def pair(idx_ref, tab_ref, n_tab, o0, o1):
            pltpu.sync_copy(idx_ref.at[pl.ds(base, SPT)], uidx)
            for g in range(SPT // 16):
                tidx[pl.ds(g * 16, 16)] = uidx[pl.ds(g * 16, 16)] + n_tab
            pltpu.async_copy(tab_ref.at[uidx], gb0, sem).wait()
            pltpu.async_copy(tab_ref.at[tidx], gb1, sem).wait()
            pltpu.sync_copy(gb0, o0.at[pl.ds(base, SPT)])
            pltpu.sync_copy(gb1, o1.at[pl.ds(base, SPT)])

        pair(uid_ref, uf_ref, NU, oU0, oU1)
        pair(iid_ref, itf_ref, NI, oA0, oA1)
        pair(iid_ref, i1h_ref, NI, oB0, oB1)
        pair(iid_ref, i1l_ref, NI, oC0, oC1)

    out = jax.ShapeDtypeStruct((B, H), jnp.float32)
    f = pl.kernel(
        body,
        out_type=(out,) * 8,
        mesh=_mesh(),
        scratch_types=[
            pltpu.VMEM((SPT,), jnp.int32),
            pltpu.VMEM((SPT,), jnp.int32),
            pltpu.VMEM((SPT, H), jnp.float32),
            pltpu.VMEM((SPT, H), jnp.float32),
            pltpu.SemaphoreType.DMA,
        ],
        compiler_params=pltpu.CompilerParams(use_tc_tiling_on_sc=False),
    )
    return f(uf, itf, i1h, i1l, uid, iid)


SBLK = 512


def _dot_body(u0, u1, a0, a1, b0, b1, c0, c1, o):
    p = (u0[...] * (a0[...] + b0[...] + c0[...])
         + u1[...] * (a1[...] + b1[...] + c1[...]))
    o[...] = jnp.sum(p, axis=1, keepdims=True)


def _scores(uf, itf, i1h, i1l, uid, iid):
    g = _gather_rows(uf, itf, i1h, i1l, uid, iid)
    spec = pl.BlockSpec((SBLK, H), lambda i: (i, 0))
    out = pl.pallas_call(
        _dot_body,
        grid=(B // SBLK,),
        in_specs=[spec] * 8,
        out_specs=pl.BlockSpec((SBLK, 1), lambda i: (i, 0)),
        out_shape=jax.ShapeDtypeStruct((B, 1), jnp.float32),
    )(*g)
    return out.reshape(B)


def kernel(user_ids, item_ids, edge_index_high, edge_vals_high, edge_index_low,
           edge_vals_low, video_emb, W_proj, b_proj, gcn_w1_high, gcn_w2_high,
           gcn_w1_low, gcn_w2_low):
    ui = user_ids.astype(jnp.int32)
    ii = item_ids.astype(jnp.int32)
    pad_i = jnp.zeros((EPAD - NNZ,), jnp.int32)
    pad_f = jnp.zeros((EPAD - NNZ,), jnp.float32)

    def pad2d(x, p):
        return jnp.concatenate([x, p]).reshape(EROWS, CHUNK)

    rh = pad2d(edge_index_high[0].astype(jnp.int32), pad_i)
    ch = pad2d(edge_index_high[1].astype(jnp.int32), pad_i)
    vh = pad2d(edge_vals_high.astype(jnp.float32), pad_f)
    rl = pad2d(edge_index_low[0].astype(jnp.int32), pad_i)
    cl = pad2d(edge_index_low[1].astype(jnp.int32), pad_i)
    vl = pad2d(edge_vals_low.astype(jnp.float32), pad_f)

    itf_s, h1h_s, h1l_s = _prep(video_emb, W_proj, b_proj.reshape(1, D),
                                gcn_w1_high, gcn_w1_low)
    itf = itf_s.reshape(2 * NI, H)

    u1h = _spmm(h1h_s.reshape(2 * NI, H), ch, rh, vh, NU)
    u1l = _spmm(h1l_s.reshape(2 * NI, H), cl, rl, vl, NU)
    i1h = _spmm(u1h, rh, ch, vh, NI)
    i1l = _spmm(u1l, rl, cl, vl, NI)

    h2h_s, h2l_s = _h2(i1h.reshape(2, NI, H), i1l.reshape(2, NI, H),
                       gcn_w2_high, gcn_w2_low)

    uf = _user_final(h2h_s.reshape(2 * NI, H), h2l_s.reshape(2 * NI, H),
                     ch, rh, vh, cl, rl, vl, u1h, u1l)

    return _scores(uf, itf, i1h, i1l, ui, ii)


# speedup vs baseline: 1.1675x; 1.1675x over previous
import functools

import jax
import jax.numpy as jnp
from jax import lax
from jax.experimental import pallas as pl
from jax.experimental.pallas import tpu as pltpu
from jax.experimental.pallas import tpu_sc as plsc

NU = 50000
NI = 50000
D = 64
H = 32
NNZ = 800000
B = 4096
NSC = 2
NTILE = 16
CHUNK = 128
BCH = 4
NBLKE = 100
NPAIR = NBLKE // 2
NCHUNK = BCH * NBLKE
EPT = CHUNK * NCHUNK
EPAD = EPT * NTILE
EROWS = EPAD // CHUNK
EXROWS = 2 * BCH
EROWS_ALLOC = EROWS + EXROWS
RCH = 128
NRCH = 391
LASTN = NU - (NRCH - 1) * RCH
LASTK = NRCH - 1
NROUND = 24

MBLK = 1000
NBLK = NI // MBLK


def _mesh():
    return plsc.VectorSubcoreMesh(core_axis_name="c", subcore_axis_name="s",
                                  num_cores=NSC, num_subcores=NTILE)


def _prep_body(vem, W, b, w1h, w1l, o_itf, o_h1h, o_h1l):
    x = vem[...]
    itf = jnp.dot(x, W[...], preferred_element_type=jnp.float32) + b[...]
    o_itf[0] = itf[:, :H]
    o_itf[1] = itf[:, H:]
    h1h = jnp.dot(itf, w1h[...], preferred_element_type=jnp.float32)
    o_h1h[0] = h1h[:, :H]
    o_h1h[1] = h1h[:, H:]
    h1l = jnp.dot(itf, w1l[...], preferred_element_type=jnp.float32)
    o_h1l[0] = h1l[:, :H]
    o_h1l[1] = h1l[:, H:]


def _prep(video_emb, W_proj, b_proj, w1h, w1l):
    out = jax.ShapeDtypeStruct((2, NI, H), jnp.float32)
    return pl.pallas_call(
        _prep_body,
        grid=(NBLK,),
        in_specs=[
            pl.BlockSpec((MBLK, D), lambda i: (i, 0)),
            pl.BlockSpec((D, D), lambda i: (0, 0)),
            pl.BlockSpec((1, D), lambda i: (0, 0)),
            pl.BlockSpec((D, D), lambda i: (0, 0)),
            pl.BlockSpec((D, D), lambda i: (0, 0)),
        ],
        out_specs=[pl.BlockSpec((2, MBLK, H), lambda i: (0, i, 0))] * 3,
        out_shape=[out, out, out],
    )(video_emb, W_proj, b_proj, w1h, w1l)


def _h2_body(ih, il, w2h, w2l, oh, ol):
    xh = jnp.concatenate([ih[0], ih[1]], axis=1)
    h2h = jnp.dot(xh, w2h[...], preferred_element_type=jnp.float32)
    oh[0] = h2h[:, :H]
    oh[1] = h2h[:, H:]
    xl = jnp.concatenate([il[0], il[1]], axis=1)
    h2l = jnp.dot(xl, w2l[...], preferred_element_type=jnp.float32)
    ol[0] = h2l[:, :H]
    ol[1] = h2l[:, H:]


def _h2(i1h, i1l, w2h, w2l):
    out = jax.ShapeDtypeStruct((2, NI, H), jnp.float32)
    return pl.pallas_call(
        _h2_body,
        grid=(NBLK,),
        in_specs=[
            pl.BlockSpec((2, MBLK, H), lambda i: (0, i, 0)),
            pl.BlockSpec((2, MBLK, H), lambda i: (0, i, 0)),
            pl.BlockSpec((D, D), lambda i: (0, 0)),
            pl.BlockSpec((D, D), lambda i: (0, 0)),
        ],
        out_specs=[pl.BlockSpec((2, MBLK, H), lambda i: (0, i, 0))] * 2,
        out_shape=[out, out],
    )(i1h, i1l, w2h, w2l)


def _for_tile_rows(s, do):
    for j in range(NROUND):
        do((s + NTILE * j) * RCH, RCH)
    rem = LASTK - NTILE * NROUND

    @pl.when(s < rem)
    def _():
        do((s + NTILE * NROUND) * RCH, RCH, tail=True)

    @pl.when(s == rem)
    def _():
        do(LASTK * RCH, LASTN, tail=True)


def _zero_acc(gbuf, acc, s, sem):
    z = jnp.zeros((16,), jnp.float32)
    for r in range(RCH):
        gbuf[r, pl.ds(0, 16)] = z
        gbuf[r, pl.ds(16, 16)] = z
    ds = []

    def do(r0, n, tail=False):
        if tail:
            pltpu.sync_copy(gbuf.at[pl.ds(0, n)], acc.at[pl.ds(r0, n)])
        else:
            ds.append(pltpu.async_copy(gbuf.at[pl.ds(0, n)],
                                       acc.at[pl.ds(r0, n)], sem))

    _for_tile_rows(s, do)
    for d in ds:
        d.wait()


def _edge_pass(tab_ref, gidx_ref, sidx_ref, vals_ref, bufs, acc, c, s, n_tab):
    stg = (bufs[0:4], bufs[4:8])
    gbs = bufs[8:8 + BCH]
    sbs = bufs[8 + BCH:10 + BCH]
    semg = bufs[10 + BCH:10 + 2 * BCH]
    sems = bufs[10 + 2 * BCH:12 + 2 * BCH]
    semb = bufs[12 + 2 * BCH:14 + 2 * BCH]
    base = c * n_tab
    row0 = s * NCHUNK

    def stage_fire(bi, p):
        r = row0 + bi * BCH
        idxb, sidb, valb, _ = stg[p]
        pltpu.async_copy(gidx_ref.at[pl.ds(r, BCH)], idxb, semb[p])
        pltpu.async_copy(sidx_ref.at[pl.ds(r, BCH)], sidb, semb[p])
        pltpu.async_copy(vals_ref.at[pl.ds(r, BCH)], valb, semb[p])

    def stage_wait(bi, p):
        r = row0 + bi * BCH
        idxb, sidb, valb, _ = stg[p]
        pltpu.make_async_copy(gidx_ref.at[pl.ds(r, BCH)], idxb, semb[p]).wait()
        pltpu.make_async_copy(sidx_ref.at[pl.ds(r, BCH)], sidb, semb[p]).wait()
        pltpu.make_async_copy(vals_ref.at[pl.ds(r, BCH)], valb, semb[p]).wait()

    def adj_compute(p):
        idxb, _, _, adjb = stg[p]
        for j in range(BCH):
            for g in range(CHUNK // 16):
                adjb[j, pl.ds(g * 16, 16)] = (idxb[j, pl.ds(g * 16, 16)]
                                              + base)

    def scale(p, j):
        _, _, valb, _ = stg[p]
        gb = gbs[j]
        for g in range(CHUNK // 16):
            vv = valb[j, pl.ds(g * 16, 16)]
            for e in range(16):
                rr = g * 16 + e
                sp = vv.at[jnp.full((16,), e, jnp.int32)].get(
                    mode="promise_in_bounds")
                sb = sbs[j % 2]
                sb[rr, pl.ds(0, 16)] = gb[rr, pl.ds(0, 16)] * sp
                sb[rr, pl.ds(16, 16)] = gb[rr, pl.ds(16, 16)] * sp

    def gather_fire(p, j):
        _, _, _, adjb = stg[p]
        return pltpu.async_copy(tab_ref.at[adjb.at[j]], gbs[j], semg[j])

    def scatter_fire(p, j):
        _, sidb, _, _ = stg[p]
        return pltpu.async_copy(sbs[j % 2], acc.at[sidb.at[j]], sems[j % 2],
                                add=True)

    stage_fire(0, 0)
    stage_fire(1, 1)

    def pair_body(pi, carry):
        b0 = 2 * pi
        stage_wait(b0, 0)
        adj_compute(0)
        d_g0 = [gather_fire(0, j) for j in range(BCH)]
        stage_wait(b0 + 1, 1)
        adj_compute(1)
        d_s0 = [None] * BCH
        d_g1 = [None] * BCH
        for j in range(BCH):
            d_g0[j].wait()
            if j >= 2:
                d_s0[j - 2].wait()
            scale(0, j)
            d_s0[j] = scatter_fire(0, j)
            d_g1[j] = gather_fire(1, j)
        d_s0[BCH - 2].wait()
        d_s0[BCH - 1].wait()
        stage_fire(b0 + 2, 0)
        d_s1 = [None] * BCH
        for j in range(BCH):
            d_g1[j].wait()
            if j >= 2:
                d_s1[j - 2].wait()
            scale(1, j)
            d_s1[j] = scatter_fire(1, j)
        d_s1[BCH - 2].wait()
        d_s1[BCH - 1].wait()
        stage_fire(b0 + 3, 1)
        return carry

    lax.fori_loop(0, NPAIR, pair_body, 0)
    stage_wait(0, 0)
    stage_wait(1, 1)


def _pass_scratch(n_out):
    stage = [pltpu.VMEM((BCH, CHUNK), jnp.int32),
             pltpu.VMEM((BCH, CHUNK), jnp.int32),
             pltpu.VMEM((BCH, CHUNK), jnp.float32),
             pltpu.VMEM((BCH, CHUNK), jnp.int32)]
    return (
        stage + stage
        + [pltpu.VMEM((CHUNK, H), jnp.float32)] * BCH
        + [pltpu.VMEM((CHUNK, H), jnp.float32)] * 2
        + [pltpu.SemaphoreType.DMA] * BCH
        + [pltpu.SemaphoreType.DMA] * 2
        + [pltpu.SemaphoreType.DMA] * 2
        + [pltpu.SemaphoreType.DMA]
        + [pltpu.VMEM_SHARED((n_out, H), jnp.float32)]
    )


def _spmm(table, gidx, sidx, vals, n_out):
    n_tab = table.shape[0] // 2

    def body(tab_ref, gidx_ref, sidx_ref, vals_ref, out_ref, *bufs):
        acc = bufs[-1]
        semb = bufs[-2]
        c = lax.axis_index("c")
        s = lax.axis_index("s")
        _zero_acc(bufs[8], acc, s, semb)
        plsc.subcore_barrier()
        _edge_pass(tab_ref, gidx_ref, sidx_ref, vals_ref, bufs[:-1], acc,
                   c, s, n_tab)
        plsc.subcore_barrier()
        ds = []

        def do(r0, n, tail=False):
            if tail:
                pltpu.sync_copy(acc.at[pl.ds(r0, n)],
                                out_ref.at[pl.ds(c * n_out + r0, n)])
            else:
                ds.append(pltpu.async_copy(
                    acc.at[pl.ds(r0, n)],
                    out_ref.at[pl.ds(c * n_out + r0, n)], semb))

        _for_tile_rows(s, do)
        for d in ds:
            d.wait()

    f = pl.kernel(
        body,
        out_type=jax.ShapeDtypeStruct((2 * n_out, H), jnp.float32),
        mesh=_mesh(),
        scratch_types=_pass_scratch(n_out),
        compiler_params=pltpu.CompilerParams(use_tc_tiling_on_sc=False),
    )
    return f(table, gidx, sidx, vals)


def _user_final(h2h, h2l, gxh, sxh, vlh, gxl, sxl, vll, u1h, u1l):
    def body(h2h_ref, h2l_ref, gxh_ref, sxh_ref, vlh_ref, gxl_ref, sxl_ref,
             vll_ref, u1h_ref, u1l_ref, out_ref, *bufs):
        acc = bufs[-1]
        semb = bufs[-2]
        gbuf, bufB, bufC = bufs[8], bufs[9], bufs[12]
        c = lax.axis_index("c")
        s = lax.axis_index("s")
        _zero_acc(gbuf, acc, s, semb)
        plsc.subcore_barrier()
        _edge_pass(h2h_ref, gxh_ref, sxh_ref, vlh_ref, bufs[:-1], acc,
                   c, s, NI)
        _edge_pass(h2l_ref, gxl_ref, sxl_ref, vll_ref, bufs[:-1], acc,
                   c, s, NI)
        plsc.subcore_barrier()

        def do(r0, n, tail=False):
            pltpu.sync_copy(acc.at[pl.ds(r0, n)], gbuf.at[pl.ds(0, n)])
            pltpu.sync_copy(u1h_ref.at[pl.ds(c * NU + r0, n)],
                            bufB.at[pl.ds(0, n)])
            pltpu.sync_copy(u1l_ref.at[pl.ds(c * NU + r0, n)],
                            bufC.at[pl.ds(0, n)])

            def addrow(i, carry):
                for j in (0, 16):
                    gbuf[i, pl.ds(j, 16)] = (gbuf[i, pl.ds(j, 16)]
                                             + bufB[i, pl.ds(j, 16)]
                                             + bufC[i, pl.ds(j, 16)])
                return carry

            lax.fori_loop(0, n, addrow, 0)
            pltpu.sync_copy(gbuf.at[pl.ds(0, n)],
                            out_ref.at[pl.ds(c * NU + r0, n)])

        _for_tile_rows(s, do)

    f = pl.kernel(
        body,
        out_type=jax.ShapeDtypeStruct((2 * NU, H), jnp.float32),
        mesh=_mesh(),
        scratch_types=_pass_scratch(NU),
        compiler_params=pltpu.CompilerParams(use_tc_tiling_on_sc=False),
    )
    return f(h2h, h2l, gxh, sxh, vlh, gxl, sxl, vll, u1h, u1l)


SPT = B // (NSC * NTILE)


def _gather_rows(uf, itf, i1h, i1l, uid, iid):
    def body(uf_ref, itf_ref, i1h_ref, i1l_ref, uid_ref, iid_ref,
             oU0, oU1, oA0, oA1, oB0, oB1, oC0, oC1,
             uidx, tidx, gb0, gb1, sem):
        c = lax.axis_index("c")
        s = lax.axis_index("s")
        wid = s * NSC + c
        base = wid * SPT

        def pair(idx_ref, tab_ref, n_tab, o0, o1):
            pltpu.sync_copy(idx_ref.at[pl.ds(base, SPT)], uidx)
            for g in range(SPT // 16):
                tidx[pl.ds(g * 16, 16)] = uidx[pl.ds(g * 16, 16)] + n_tab
            pltpu.async_copy(tab_ref.at[uidx], gb0, sem).wait()
            pltpu.async_copy(tab_ref.at[tidx], gb1, sem).wait()
            pltpu.sync_copy(gb0, o0.at[pl.ds(base, SPT)])
            pltpu.sync_copy(gb1, o1.at[pl.ds(base, SPT)])

        pair(uid_ref, uf_ref, NU, oU0, oU1)
        pair(iid_ref, itf_ref, NI, oA0, oA1)
        pair(iid_ref, i1h_ref, NI, oB0, oB1)
        pair(iid_ref, i1l_ref, NI, oC0, oC1)

    out = jax.ShapeDtypeStruct((B, H), jnp.float32)
    f = pl.kernel(
        body,
        out_type=(out,) * 8,
        mesh=_mesh(),
        scratch_types=[
            pltpu.VMEM((SPT,), jnp.int32),
            pltpu.VMEM((SPT,), jnp.int32),
            pltpu.VMEM((SPT, H), jnp.float32),
            pltpu.VMEM((SPT, H), jnp.float32),
            pltpu.SemaphoreType.DMA,
        ],
        compiler_params=pltpu.CompilerParams(use_tc_tiling_on_sc=False),
    )
    return f(uf, itf, i1h, i1l, uid, iid)


SBLK = 512


def _dot_body(u0, u1, a0, a1, b0, b1, c0, c1, o):
    p = (u0[...] * (a0[...] + b0[...] + c0[...])
         + u1[...] * (a1[...] + b1[...] + c1[...]))
    o[...] = jnp.sum(p, axis=1, keepdims=True)


def _scores(uf, itf, i1h, i1l, uid, iid):
    g = _gather_rows(uf, itf, i1h, i1l, uid, iid)
    spec = pl.BlockSpec((SBLK, H), lambda i: (i, 0))
    out = pl.pallas_call(
        _dot_body,
        grid=(B // SBLK,),
        in_specs=[spec] * 8,
        out_specs=pl.BlockSpec((SBLK, 1), lambda i: (i, 0)),
        out_shape=jax.ShapeDtypeStruct((B, 1), jnp.float32),
    )(*g)
    return out.reshape(B)


def kernel(user_ids, item_ids, edge_index_high, edge_vals_high, edge_index_low,
           edge_vals_low, video_emb, W_proj, b_proj, gcn_w1_high, gcn_w2_high,
           gcn_w1_low, gcn_w2_low):
    ui = user_ids.astype(jnp.int32)
    ii = item_ids.astype(jnp.int32)
    npad = EROWS_ALLOC * CHUNK - NNZ
    pad_i = jnp.zeros((npad,), jnp.int32)
    pad_f = jnp.zeros((npad,), jnp.float32)

    def pad2d(x, p):
        return jnp.concatenate([x, p]).reshape(EROWS_ALLOC, CHUNK)

    rh = pad2d(edge_index_high[0].astype(jnp.int32), pad_i)
    ch = pad2d(edge_index_high[1].astype(jnp.int32), pad_i)
    vh = pad2d(edge_vals_high.astype(jnp.float32), pad_f)
    rl = pad2d(edge_index_low[0].astype(jnp.int32), pad_i)
    cl = pad2d(edge_index_low[1].astype(jnp.int32), pad_i)
    vl = pad2d(edge_vals_low.astype(jnp.float32), pad_f)

    itf_s, h1h_s, h1l_s = _prep(video_emb, W_proj, b_proj.reshape(1, D),
                                gcn_w1_high, gcn_w1_low)
    itf = itf_s.reshape(2 * NI, H)

    u1h = _spmm(h1h_s.reshape(2 * NI, H), ch, rh, vh, NU)
    u1l = _spmm(h1l_s.reshape(2 * NI, H), cl, rl, vl, NU)
    i1h = _spmm(u1h, rh, ch, vh, NI)
    i1l = _spmm(u1l, rl, cl, vl, NI)

    h2h_s, h2l_s = _h2(i1h.reshape(2, NI, H), i1l.reshape(2, NI, H),
                       gcn_w2_high, gcn_w2_low)

    uf = _user_final(h2h_s.reshape(2 * NI, H), h2l_s.reshape(2 * NI, H),
                     ch, rh, vh, cl, rl, vl, u1h, u1l)

    return _scores(uf, itf, i1h, i1l, ui, ii)


# speedup vs baseline: 1.1754x; 1.0068x over previous
import functools

import jax
import jax.numpy as jnp
from jax import lax
from jax.experimental import pallas as pl
from jax.experimental.pallas import tpu as pltpu
from jax.experimental.pallas import tpu_sc as plsc

NU = 50000
NI = 50000
D = 64
H = 32
NNZ = 800000
B = 4096
NSC = 2
NTILE = 16
CHUNK = 128
BCH = 4
NBLKE = 100
NPAIR = NBLKE // 2
NCHUNK = BCH * NBLKE
EPT = CHUNK * NCHUNK
EPAD = EPT * NTILE
EROWS = EPAD // CHUNK
EXROWS = 2 * BCH
EROWS_ALLOC = EROWS + EXROWS
RCH = 128
NRCH = 391
LASTN = NU - (NRCH - 1) * RCH
LASTK = NRCH - 1
NROUND = 24

MBLK = 1000
NBLK = NI // MBLK


def _mesh():
    return plsc.VectorSubcoreMesh(core_axis_name="c", subcore_axis_name="s",
                                  num_cores=NSC, num_subcores=NTILE)


def _prep_body(vem, W, b, w1h, w1l, o_itf, o_h1h, o_h1l):
    x = vem[...]
    itf = jnp.dot(x, W[...], preferred_element_type=jnp.float32) + b[...]
    o_itf[0] = itf[:, :H]
    o_itf[1] = itf[:, H:]
    h1h = jnp.dot(itf, w1h[...], preferred_element_type=jnp.float32)
    o_h1h[0] = h1h[:, :H]
    o_h1h[1] = h1h[:, H:]
    h1l = jnp.dot(itf, w1l[...], preferred_element_type=jnp.float32)
    o_h1l[0] = h1l[:, :H]
    o_h1l[1] = h1l[:, H:]


def _prep(video_emb, W_proj, b_proj, w1h, w1l):
    out = jax.ShapeDtypeStruct((2, NI, H), jnp.float32)
    return pl.pallas_call(
        _prep_body,
        grid=(NBLK,),
        in_specs=[
            pl.BlockSpec((MBLK, D), lambda i: (i, 0)),
            pl.BlockSpec((D, D), lambda i: (0, 0)),
            pl.BlockSpec((1, D), lambda i: (0, 0)),
            pl.BlockSpec((D, D), lambda i: (0, 0)),
            pl.BlockSpec((D, D), lambda i: (0, 0)),
        ],
        out_specs=[pl.BlockSpec((2, MBLK, H), lambda i: (0, i, 0))] * 3,
        out_shape=[out, out, out],
    )(video_emb, W_proj, b_proj, w1h, w1l)


def _h2_body(ih, il, w2h, w2l, oh, ol):
    xh = jnp.concatenate([ih[0], ih[1]], axis=1)
    h2h = jnp.dot(xh, w2h[...], preferred_element_type=jnp.float32)
    oh[0] = h2h[:, :H]
    oh[1] = h2h[:, H:]
    xl = jnp.concatenate([il[0], il[1]], axis=1)
    h2l = jnp.dot(xl, w2l[...], preferred_element_type=jnp.float32)
    ol[0] = h2l[:, :H]
    ol[1] = h2l[:, H:]


def _h2(i1h, i1l, w2h, w2l):
    out = jax.ShapeDtypeStruct((2, NI, H), jnp.float32)
    return pl.pallas_call(
        _h2_body,
        grid=(NBLK,),
        in_specs=[
            pl.BlockSpec((2, MBLK, H), lambda i: (0, i, 0)),
            pl.BlockSpec((2, MBLK, H), lambda i: (0, i, 0)),
            pl.BlockSpec((D, D), lambda i: (0, 0)),
            pl.BlockSpec((D, D), lambda i: (0, 0)),
        ],
        out_specs=[pl.BlockSpec((2, MBLK, H), lambda i: (0, i, 0))] * 2,
        out_shape=[out, out],
    )(i1h, i1l, w2h, w2l)


def _for_tile_rows(s, do):
    for j in range(NROUND):
        do((s + NTILE * j) * RCH, RCH)
    rem = LASTK - NTILE * NROUND

    @pl.when(s < rem)
    def _():
        do((s + NTILE * NROUND) * RCH, RCH, tail=True)

    @pl.when(s == rem)
    def _():
        do(LASTK * RCH, LASTN, tail=True)


def _zero_acc(gbuf, acc, s, sem):
    z = jnp.zeros((16,), jnp.float32)
    for r in range(RCH):
        gbuf[r, pl.ds(0, 16)] = z
        gbuf[r, pl.ds(16, 16)] = z
    ds = []

    def do(r0, n, tail=False):
        if tail:
            pltpu.sync_copy(gbuf.at[pl.ds(0, n)], acc.at[pl.ds(r0, n)])
        else:
            ds.append(pltpu.async_copy(gbuf.at[pl.ds(0, n)],
                                       acc.at[pl.ds(r0, n)], sem))

    _for_tile_rows(s, do)
    for d in ds:
        d.wait()


def _edge_pass(tab_ref, gidx_ref, sv_ref, bufs, acc, c, s, n_tab):
    stg = (bufs[0:2], bufs[2:4])
    gbs = bufs[4:4 + BCH]
    sbs = bufs[4 + BCH:6 + BCH]
    semg = bufs[6 + BCH:6 + 2 * BCH]
    sems = bufs[6 + 2 * BCH:8 + 2 * BCH]
    semb = bufs[8 + 2 * BCH:10 + 2 * BCH]
    del n_tab
    row0 = s * NCHUNK

    def stage_fire(bi, p):
        r = row0 + bi * BCH
        idxb, svb = stg[p]
        pltpu.async_copy(gidx_ref.at[c, pl.ds(r, BCH)], idxb, semb[p])
        pltpu.async_copy(sv_ref.at[pl.ds(r, BCH)], svb, semb[p])

    def stage_wait(bi, p):
        r = row0 + bi * BCH
        idxb, svb = stg[p]
        pltpu.make_async_copy(gidx_ref.at[c, pl.ds(r, BCH)], idxb,
                              semb[p]).wait()
        pltpu.make_async_copy(sv_ref.at[pl.ds(r, BCH)], svb, semb[p]).wait()

    def adj_compute(p):
        pass

    def scale(p, j):
        _, svb = stg[p]
        gb = gbs[j]
        for g in range(CHUNK // 16):
            vv = lax.bitcast_convert_type(svb[j, 1, pl.ds(g * 16, 16)],
                                          jnp.float32)
            for e in range(16):
                rr = g * 16 + e
                sp = vv.at[jnp.full((16,), e, jnp.int32)].get(
                    mode="promise_in_bounds")
                sb = sbs[j % 2]
                sb[rr, pl.ds(0, 16)] = gb[rr, pl.ds(0, 16)] * sp
                sb[rr, pl.ds(16, 16)] = gb[rr, pl.ds(16, 16)] * sp

    def gather_fire(p, j):
        idxb, _ = stg[p]
        return pltpu.async_copy(tab_ref.at[idxb.at[j]], gbs[j], semg[j])

    def scatter_fire(p, j):
        _, svb = stg[p]
        return pltpu.async_copy(sbs[j % 2], acc.at[svb.at[j, 0]],
                                sems[j % 2], add=True)

    stage_fire(0, 0)
    stage_fire(1, 1)

    def pair_body(pi, carry):
        b0 = 2 * pi
        stage_wait(b0, 0)
        adj_compute(0)
        d_g0 = [gather_fire(0, j) for j in range(BCH)]
        stage_wait(b0 + 1, 1)
        adj_compute(1)
        d_s0 = [None] * BCH
        d_g1 = [None] * BCH
        for j in range(BCH):
            d_g0[j].wait()
            if j >= 2:
                d_s0[j - 2].wait()
            scale(0, j)
            d_s0[j] = scatter_fire(0, j)
            d_g1[j] = gather_fire(1, j)
        d_s0[BCH - 2].wait()
        d_s0[BCH - 1].wait()
        stage_fire(b0 + 2, 0)
        d_s1 = [None] * BCH
        for j in range(BCH):
            d_g1[j].wait()
            if j >= 2:
                d_s1[j - 2].wait()
            scale(1, j)
            d_s1[j] = scatter_fire(1, j)
        d_s1[BCH - 2].wait()
        d_s1[BCH - 1].wait()
        stage_fire(b0 + 3, 1)
        return carry

    lax.fori_loop(0, NPAIR, pair_body, 0)
    stage_wait(0, 0)
    stage_wait(1, 1)


def _pass_scratch(n_out):
    stage = [pltpu.VMEM((BCH, CHUNK), jnp.int32),
             pltpu.VMEM((BCH, 2, CHUNK), jnp.int32)]
    return (
        stage + stage
        + [pltpu.VMEM((CHUNK, H), jnp.float32)] * BCH
        + [pltpu.VMEM((CHUNK, H), jnp.float32)] * 2
        + [pltpu.SemaphoreType.DMA] * BCH
        + [pltpu.SemaphoreType.DMA] * 2
        + [pltpu.SemaphoreType.DMA] * 2
        + [pltpu.SemaphoreType.DMA]
        + [pltpu.VMEM_SHARED((n_out, H), jnp.float32)]
    )


def _spmm(table, gidxp, sv, n_out):
    def body(tab_ref, gidx_ref, sv_ref, out_ref, *bufs):
        acc = bufs[-1]
        semb = bufs[-2]
        c = lax.axis_index("c")
        s = lax.axis_index("s")
        _zero_acc(bufs[4], acc, s, semb)
        plsc.subcore_barrier()
        _edge_pass(tab_ref, gidx_ref, sv_ref, bufs[:-1], acc, c, s, 0)
        plsc.subcore_barrier()
        ds = []

        def do(r0, n, tail=False):
            if tail:
                pltpu.sync_copy(acc.at[pl.ds(r0, n)],
                                out_ref.at[pl.ds(c * n_out + r0, n)])
            else:
                ds.append(pltpu.async_copy(
                    acc.at[pl.ds(r0, n)],
                    out_ref.at[pl.ds(c * n_out + r0, n)], semb))

        _for_tile_rows(s, do)
        for d in ds:
            d.wait()

    f = pl.kernel(
        body,
        out_type=jax.ShapeDtypeStruct((2 * n_out, H), jnp.float32),
        mesh=_mesh(),
        scratch_types=_pass_scratch(n_out),
        compiler_params=pltpu.CompilerParams(use_tc_tiling_on_sc=False),
    )
    return f(table, gidxp, sv)


def _user_final(h2h, h2l, gch, svh, gcl, svl, u1h, u1l):
    def body(h2h_ref, h2l_ref, gch_ref, svh_ref, gcl_ref, svl_ref,
             u1h_ref, u1l_ref, out_ref, *bufs):
        acc = bufs[-1]
        semb = bufs[-2]
        gbuf, bufB, bufC = bufs[4], bufs[5], bufs[8]
        c = lax.axis_index("c")
        s = lax.axis_index("s")
        _zero_acc(gbuf, acc, s, semb)
        plsc.subcore_barrier()
        _edge_pass(h2h_ref, gch_ref, svh_ref, bufs[:-1], acc, c, s, 0)
        _edge_pass(h2l_ref, gcl_ref, svl_ref, bufs[:-1], acc, c, s, 0)
        plsc.subcore_barrier()

        def do(r0, n, tail=False):
            pltpu.sync_copy(acc.at[pl.ds(r0, n)], gbuf.at[pl.ds(0, n)])
            pltpu.sync_copy(u1h_ref.at[pl.ds(c * NU + r0, n)],
                            bufB.at[pl.ds(0, n)])
            pltpu.sync_copy(u1l_ref.at[pl.ds(c * NU + r0, n)],
                            bufC.at[pl.ds(0, n)])

            def addrow(i, carry):
                for j in (0, 16):
                    gbuf[i, pl.ds(j, 16)] = (gbuf[i, pl.ds(j, 16)]
                                             + bufB[i, pl.ds(j, 16)]
                                             + bufC[i, pl.ds(j, 16)])
                return carry

            lax.fori_loop(0, n, addrow, 0)
            pltpu.sync_copy(gbuf.at[pl.ds(0, n)],
                            out_ref.at[pl.ds(c * NU + r0, n)])

        _for_tile_rows(s, do)

    f = pl.kernel(
        body,
        out_type=jax.ShapeDtypeStruct((2 * NU, H), jnp.float32),
        mesh=_mesh(),
        scratch_types=_pass_scratch(NU),
        compiler_params=pltpu.CompilerParams(use_tc_tiling_on_sc=False),
    )
    return f(h2h, h2l, gch, svh, gcl, svl, u1h, u1l)


SPT = B // (NSC * NTILE)


def _gather_rows(uf, itf, i1h, i1l, uid, iid):
    def body(uf_ref, itf_ref, i1h_ref, i1l_ref, uid_ref, iid_ref,
             oU0, oU1, oA0, oA1, oB0, oB1, oC0, oC1,
             uidx, tidx, gb0, gb1, sem):
        c = lax.axis_index("c")
        s = lax.axis_index("s")
        wid = s * NSC + c
        base = wid * SPT

        def pair(idx_ref, tab_ref, n_tab, o0, o1):
            pltpu.sync_copy(idx_ref.at[pl.ds(base, SPT)], uidx)
            for g in range(SPT // 16):
                tidx[pl.ds(g * 16, 16)] = uidx[pl.ds(g * 16, 16)] + n_tab
            pltpu.async_copy(tab_ref.at[uidx], gb0, sem).wait()
            pltpu.async_copy(tab_ref.at[tidx], gb1, sem).wait()
            pltpu.sync_copy(gb0, o0.at[pl.ds(base, SPT)])
            pltpu.sync_copy(gb1, o1.at[pl.ds(base, SPT)])

        pair(uid_ref, uf_ref, NU, oU0, oU1)
        pair(iid_ref, itf_ref, NI, oA0, oA1)
        pair(iid_ref, i1h_ref, NI, oB0, oB1)
        pair(iid_ref, i1l_ref, NI, oC0, oC1)

    out = jax.ShapeDtypeStruct((B, H), jnp.float32)
    f = pl.kernel(
        body,
        out_type=(out,) * 8,
        mesh=_mesh(),
        scratch_types=[
            pltpu.VMEM((SPT,), jnp.int32),
            pltpu.VMEM((SPT,), jnp.int32),
            pltpu.VMEM((SPT, H), jnp.float32),
            pltpu.VMEM((SPT, H), jnp.float32),
            pltpu.SemaphoreType.DMA,
        ],
        compiler_params=pltpu.CompilerParams(use_tc_tiling_on_sc=False),
    )
    return f(uf, itf, i1h, i1l, uid, iid)


SBLK = 512


def _dot_body(u0, u1, a0, a1, b0, b1, c0, c1, o):
    p = (u0[...] * (a0[...] + b0[...] + c0[...])
         + u1[...] * (a1[...] + b1[...] + c1[...]))
    o[...] = jnp.sum(p, axis=1, keepdims=True)


def _scores(uf, itf, i1h, i1l, uid, iid):
    g = _gather_rows(uf, itf, i1h, i1l, uid, iid)
    spec = pl.BlockSpec((SBLK, H), lambda i: (i, 0))
    out = pl.pallas_call(
        _dot_body,
        grid=(B // SBLK,),
        in_specs=[spec] * 8,
        out_specs=pl.BlockSpec((SBLK, 1), lambda i: (i, 0)),
        out_shape=jax.ShapeDtypeStruct((B, 1), jnp.float32),
    )(*g)
    return out.reshape(B)


def kernel(user_ids, item_ids, edge_index_high, edge_vals_high, edge_index_low,
           edge_vals_low, video_emb, W_proj, b_proj, gcn_w1_high, gcn_w2_high,
           gcn_w1_low, gcn_w2_low):
    ui = user_ids.astype(jnp.int32)
    ii = item_ids.astype(jnp.int32)
    npad = EROWS_ALLOC * CHUNK - NNZ
    pad_i = jnp.zeros((npad,), jnp.int32)

    def pad2d(x):
        return jnp.concatenate([x.astype(jnp.int32), pad_i]).reshape(
            EROWS_ALLOC, CHUNK)

    def planes(x2d, n_tab):
        return jnp.stack([x2d, x2d + n_tab])

    def packsv(sidx2d, vbits2d):
        return jnp.stack([sidx2d, vbits2d], axis=1)

    rh = pad2d(edge_index_high[0])
    ch = pad2d(edge_index_high[1])
    vhb = pad2d(lax.bitcast_convert_type(edge_vals_high.astype(jnp.float32),
                                         jnp.int32))
    rl = pad2d(edge_index_low[0])
    cl = pad2d(edge_index_low[1])
    vlb = pad2d(lax.bitcast_convert_type(edge_vals_low.astype(jnp.float32),
                                         jnp.int32))
    gch, grh = planes(ch, NI), planes(rh, NU)
    gcl, grl = planes(cl, NI), planes(rl, NU)
    sv_rh, sv_ch = packsv(rh, vhb), packsv(ch, vhb)
    sv_rl, sv_cl = packsv(rl, vlb), packsv(cl, vlb)

    itf_s, h1h_s, h1l_s = _prep(video_emb, W_proj, b_proj.reshape(1, D),
                                gcn_w1_high, gcn_w1_low)
    itf = itf_s.reshape(2 * NI, H)

    u1h = _spmm(h1h_s.reshape(2 * NI, H), gch, sv_rh, NU)
    u1l = _spmm(h1l_s.reshape(2 * NI, H), gcl, sv_rl, NU)
    i1h = _spmm(u1h, grh, sv_ch, NI)
    i1l = _spmm(u1l, grl, sv_cl, NI)

    h2h_s, h2l_s = _h2(i1h.reshape(2, NI, H), i1l.reshape(2, NI, H),
                       gcn_w2_high, gcn_w2_low)

    uf = _user_final(h2h_s.reshape(2 * NI, H), h2l_s.reshape(2 * NI, H),
                     gch, sv_rh, gcl, sv_rl, u1h, u1l)

    return _scores(uf, itf, i1h, i1l, ui, ii)
